# baseline (speedup 1.0000x reference)
import functools

import jax
import jax.numpy as jnp
from jax import lax
from jax.experimental import pallas as pl
from jax.experimental.pallas import tpu as pltpu
from jax.experimental.pallas import tpu_sc as plsc

N_NODES = 10000
NP = 10112
D = 128
H = 128
C = 40
CP = 48
E = 320000
CHUNK = 128
NC, NS = 2, 16
NW = NC * NS
KCH = 80
EP = NW * CHUNK * KCH
ECH = EP // CHUNK
RPT = NP // NS
ZR = 64
DW = 16
RB = 632
K_SLOW = 32
K_FAST = 2 * KCH - K_SLOW
SLOW_CORE = 0
SMAX = K_FAST // 2

_MESH = dict(core_axis_name="c", subcore_axis_name="s")


def _seg_sum_kernel(width, tc_tiling=True):
    @functools.partial(
        pl.kernel,
        out_type=jax.ShapeDtypeStruct((NC * NP, width), jnp.float32),
        mesh=plsc.VectorSubcoreMesh(**_MESH),
        compiler_params=pltpu.CompilerParams(use_tc_tiling_on_sc=tc_tiling),
        scratch_types=[
            pltpu.VMEM((SMAX, CHUNK), jnp.int32),
            pltpu.VMEM((SMAX, CHUNK), jnp.int32),
            pltpu.VMEM((CHUNK, width), jnp.float32),
            pltpu.VMEM((CHUNK, width), jnp.float32),
            pltpu.VMEM_SHARED((NP, width), jnp.float32),
            pltpu.SemaphoreType.DMA,
            pltpu.SemaphoreType.DMA,
        ],
    )
    def k(g_hbm, src_hbm, dst_hbm, out_hbm, src_v, dst_v, buf0, buf1, acc,
          gsem0, gsem1):
        cid = lax.axis_index("c")
        sid = lax.axis_index("s")
        bufs = (buf0, buf1)
        gsems = (gsem0, gsem1)

        slow = cid == SLOW_CORE
        count = jnp.where(slow, K_SLOW, K_FAST)
        scount = count // 2
        tile_base = jnp.where(slow, sid * K_SLOW,
                              NS * K_SLOW + sid * K_FAST)

        def start_gather(b, j):
            pltpu.async_copy(g_hbm.at[src_v.at[j]], bufs[b], gsems[b])

        def wait_gather(b, j):
            pltpu.make_async_copy(g_hbm.at[src_v.at[j]], bufs[b],
                                  gsems[b]).wait()

        def zfill(i, _):
            r = i // (width // 16)
            c = lax.rem(i, width // 16)
            buf0[r, pl.ds(c * 16, 16)] = jnp.zeros((16,), jnp.float32)
            return 0

        lax.fori_loop(0, CHUNK * (width // 16), zfill, 0)

        for r in range(RPT // CHUNK):
            pltpu.sync_copy(buf0, acc.at[pl.ds(sid * RPT + r * CHUNK, CHUNK)])
        rem = RPT - (RPT // CHUNK) * CHUNK
        if rem:
            pltpu.sync_copy(
                buf0.at[pl.ds(0, rem)],
                acc.at[pl.ds(sid * RPT + RPT - rem, rem)])
        plsc.subcore_barrier()

        for s in range(2):
            base = pl.multiple_of(tile_base + s * scount, 8)
            pltpu.sync_copy(src_hbm.at[pl.ds(base, SMAX)], src_v)
            pltpu.sync_copy(dst_hbm.at[pl.ds(base, SMAX)], dst_v)

            start_gather(0, 0)

            def body(i, _):
                for b in range(2):
                    j = 2 * i + b
                    jn = j + 1
                    bn = 1 - b

                    @pl.when(jn < scount)
                    def _():
                        start_gather(bn, jn)

                    wait_gather(b, j)
                    pltpu.sync_copy(bufs[b], acc.at[dst_v.at[j]], add=True)
                return 0

            lax.fori_loop(0, scount // 2, body, 0)

        plsc.subcore_barrier()
        pltpu.sync_copy(acc.at[pl.ds(sid * RPT, RPT)],
                        out_hbm.at[pl.ds(cid * NP + sid * RPT, RPT)])

    return k


@functools.partial(
    pl.kernel,
    out_type=jax.ShapeDtypeStruct((NC * NP, DW), jnp.float32),
    mesh=plsc.VectorSubcoreMesh(**_MESH),
    scratch_types=[
        pltpu.VMEM((KCH, CHUNK), jnp.int32),
        pltpu.VMEM((CHUNK, DW), jnp.float32),
        pltpu.VMEM((ZR, DW), jnp.float32),
        pltpu.VMEM_SHARED((NP, DW), jnp.float32),
        pltpu.SemaphoreType.DMA,
    ],
)
def _deg_kernel(dst_hbm, out_hbm, dst_v, ones_v, zbuf, acc, sem):
    cid = lax.axis_index("c")
    sid = lax.axis_index("s")
    tid = cid * NS + sid
    pltpu.sync_copy(dst_hbm.at[pl.ds(tid * KCH, KCH)], dst_v)

    def ofill(i, _):
        ones_v[i, pl.ds(0, 16)] = jnp.ones((16,), jnp.float32)
        return 0

    lax.fori_loop(0, CHUNK, ofill, 0)

    def zfill(i, _):
        zbuf[i, pl.ds(0, 16)] = jnp.zeros((16,), jnp.float32)
        return 0

    lax.fori_loop(0, ZR, zfill, 0)

    for r in range(RPT // ZR):
        pltpu.sync_copy(zbuf, acc.at[pl.ds(sid * RPT + r * ZR, ZR)])
    drem = RPT - (RPT // ZR) * ZR
    if drem:
        pltpu.sync_copy(zbuf.at[pl.ds(0, drem)],
                        acc.at[pl.ds(sid * RPT + RPT - drem, drem)])
    plsc.subcore_barrier()

    def body(j, _):
        pltpu.sync_copy(ones_v, acc.at[dst_v.at[j]], add=True)
        return 0

    lax.fori_loop(0, KCH, body, 0)
    plsc.subcore_barrier()
    pltpu.sync_copy(acc.at[pl.ds(sid * RPT, RPT)],
                    out_hbm.at[pl.ds(cid * NP + sid * RPT, RPT)])


def _dinv_block(dr):
    deg = dr[0, :, 0:1] + dr[1, :, 0:1] + 1.0
    return lax.rsqrt(deg)


def _g1_body(xr, wr, dr, gr):
    dinv = _dinv_block(dr)
    hw = jnp.dot(xr[...], wr[...], preferred_element_type=jnp.float32,
                 precision=lax.Precision.HIGHEST)
    gr[...] = hw * dinv


def _hidden_body(sr, gr, dr, wr, b1r, scr, bir, outr):
    dinv = _dinv_block(dr)
    agg = (sr[0] + sr[1] + gr[...]) * dinv
    h = (agg + b1r[...]) * scr[...] + bir[...]
    h = jnp.maximum(h, 0.0)
    outr[...] = jnp.dot(h, wr[...], preferred_element_type=jnp.float32,
                        precision=lax.Precision.HIGHEST) * dinv


def _out_body(sr, gr, dr, br, outr):
    dinv = _dinv_block(dr)
    outr[...] = (sr[0] + sr[1] + gr[...]) * dinv + br[...]


def _row_spec(w):
    return pl.BlockSpec((RB, w), lambda i: (i, 0))


def _part_spec(w):
    return pl.BlockSpec((NC, RB, w), lambda i: (0, i, 0))


def _full_spec(shape):
    return pl.BlockSpec(shape, lambda i: tuple(0 for _ in shape))


_GRID = (NP // RB,)

_g1_call = pl.pallas_call(
    _g1_body,
    grid=_GRID,
    in_specs=[_row_spec(D), _full_spec((D, H)), _part_spec(DW)],
    out_specs=_row_spec(H),
    out_shape=jax.ShapeDtypeStruct((NP, H), jnp.float32),
)

_hidden_call = pl.pallas_call(
    _hidden_body,
    grid=_GRID,
    in_specs=[_part_spec(H), _row_spec(H), _part_spec(DW),
              _full_spec((H, CP)), _full_spec((1, H)), _full_spec((1, H)),
              _full_spec((1, H))],
    out_specs=_row_spec(CP),
    out_shape=jax.ShapeDtypeStruct((NP, CP), jnp.float32),
)

_out_call = pl.pallas_call(
    _out_body,
    grid=_GRID,
    in_specs=[_part_spec(CP), _row_spec(CP), _part_spec(DW),
              _full_spec((1, CP))],
    out_specs=_row_spec(CP),
    out_shape=jax.ShapeDtypeStruct((NP, CP), jnp.float32),
)

_seg_h = _seg_sum_kernel(H)
_seg_c = _seg_sum_kernel(CP, tc_tiling=False)


def kernel(x, edge_index, W1, b1, bn1_scale, bn1_bias, W2, b2):
    src = edge_index[0].astype(jnp.int32)
    dst = edge_index[1].astype(jnp.int32)
    fill = jnp.full((EP - E,), N_NODES, jnp.int32)
    src_p = jnp.concatenate([src, fill]).reshape(ECH, CHUNK)
    dst_p = jnp.concatenate([dst, fill]).reshape(ECH, CHUNK)
    x_p = jnp.pad(x, ((0, NP - N_NODES), (0, 0)))
    W2p = jnp.pad(W2, ((0, 0), (0, CP - C)))
    b2p = jnp.pad(b2, ((0, CP - C),)).reshape(1, CP)
    b1r = b1.reshape(1, H)
    scr = bn1_scale.reshape(1, H)
    bir = bn1_bias.reshape(1, H)

    degp = _deg_kernel(dst_p).reshape(NC, NP, DW)
    g1 = _g1_call(x_p, W1, degp)
    s1 = _seg_h(g1, src_p, dst_p).reshape(NC, NP, H)
    g2 = _hidden_call(s1, g1, degp, W2p, b1r, scr, bir)
    s2 = _seg_c(g2, src_p, dst_p).reshape(NC, NP, CP)
    outp = _out_call(s2, g2, degp, b2p)
    return outp[:N_NODES, :C]


# speedup vs baseline: 1.0218x; 1.0218x over previous
import functools

import jax
import jax.numpy as jnp
from jax import lax
from jax.experimental import pallas as pl
from jax.experimental.pallas import tpu as pltpu
from jax.experimental.pallas import tpu_sc as plsc

N_NODES = 10000
NP = 10112
D = 128
H = 128
C = 40
CP = 48
E = 320000
CHUNK = 128
NC, NS = 2, 16
NW = NC * NS
KCH = 80
EP = NW * CHUNK * KCH
ECH = EP // CHUNK
RPT = NP // NS
ZR = 64
DW = 16
RB = 632
K_SLOW = 32
K_FAST = 2 * KCH - K_SLOW
SLOW_CORE = 1
SMAX = K_FAST // 2

_MESH = dict(core_axis_name="c", subcore_axis_name="s")


def _seg_sum_kernel(width, tc_tiling=True):
    @functools.partial(
        pl.kernel,
        out_type=jax.ShapeDtypeStruct((NC * NP, width), jnp.float32),
        mesh=plsc.VectorSubcoreMesh(**_MESH),
        compiler_params=pltpu.CompilerParams(use_tc_tiling_on_sc=tc_tiling),
        scratch_types=[
            pltpu.VMEM((SMAX, CHUNK), jnp.int32),
            pltpu.VMEM((SMAX, CHUNK), jnp.int32),
            pltpu.VMEM((CHUNK, width), jnp.float32),
            pltpu.VMEM((CHUNK, width), jnp.float32),
            pltpu.VMEM_SHARED((NP, width), jnp.float32),
            pltpu.SemaphoreType.DMA,
            pltpu.SemaphoreType.DMA,
        ],
    )
    def k(g_hbm, src_hbm, dst_hbm, out_hbm, src_v, dst_v, buf0, buf1, acc,
          gsem0, gsem1):
        cid = lax.axis_index("c")
        sid = lax.axis_index("s")
        bufs = (buf0, buf1)
        gsems = (gsem0, gsem1)

        slow = cid == SLOW_CORE
        count = jnp.where(slow, K_SLOW, K_FAST)
        scount = count // 2
        tile_base = jnp.where(slow, sid * K_SLOW,
                              NS * K_SLOW + sid * K_FAST)

        def start_gather(b, j):
            pltpu.async_copy(g_hbm.at[src_v.at[j]], bufs[b], gsems[b])

        def wait_gather(b, j):
            pltpu.make_async_copy(g_hbm.at[src_v.at[j]], bufs[b],
                                  gsems[b]).wait()

        def zfill(i, _):
            r = i // (width // 16)
            c = lax.rem(i, width // 16)
            buf0[r, pl.ds(c * 16, 16)] = jnp.zeros((16,), jnp.float32)
            return 0

        lax.fori_loop(0, CHUNK * (width // 16), zfill, 0)

        for r in range(RPT // CHUNK):
            pltpu.sync_copy(buf0, acc.at[pl.ds(sid * RPT + r * CHUNK, CHUNK)])
        rem = RPT - (RPT // CHUNK) * CHUNK
        if rem:
            pltpu.sync_copy(
                buf0.at[pl.ds(0, rem)],
                acc.at[pl.ds(sid * RPT + RPT - rem, rem)])
        plsc.subcore_barrier()

        for s in range(2):
            base = pl.multiple_of(tile_base + s * scount, 8)
            pltpu.sync_copy(src_hbm.at[pl.ds(base, SMAX)], src_v)
            pltpu.sync_copy(dst_hbm.at[pl.ds(base, SMAX)], dst_v)

            start_gather(0, 0)

            def body(i, _):
                for b in range(2):
                    j = 2 * i + b
                    jn = j + 1
                    bn = 1 - b

                    @pl.when(jn < scount)
                    def _():
                        start_gather(bn, jn)

                    wait_gather(b, j)
                    pltpu.sync_copy(bufs[b], acc.at[dst_v.at[j]], add=True)
                return 0

            lax.fori_loop(0, scount // 2, body, 0)

        plsc.subcore_barrier()
        pltpu.sync_copy(acc.at[pl.ds(sid * RPT, RPT)],
                        out_hbm.at[pl.ds(cid * NP + sid * RPT, RPT)])

    return k


@functools.partial(
    pl.kernel,
    out_type=jax.ShapeDtypeStruct((NC * NP, DW), jnp.float32),
    mesh=plsc.VectorSubcoreMesh(**_MESH),
    scratch_types=[
        pltpu.VMEM((KCH, CHUNK), jnp.int32),
        pltpu.VMEM((CHUNK, DW), jnp.float32),
        pltpu.VMEM((ZR, DW), jnp.float32),
        pltpu.VMEM_SHARED((NP, DW), jnp.float32),
        pltpu.SemaphoreType.DMA,
    ],
)
def _deg_kernel(dst_hbm, out_hbm, dst_v, ones_v, zbuf, acc, sem):
    cid = lax.axis_index("c")
    sid = lax.axis_index("s")
    tid = cid * NS + sid
    pltpu.sync_copy(dst_hbm.at[pl.ds(tid * KCH, KCH)], dst_v)

    def ofill(i, _):
        ones_v[i, pl.ds(0, 16)] = jnp.ones((16,), jnp.float32)
        return 0

    lax.fori_loop(0, CHUNK, ofill, 0)

    def zfill(i, _):
        zbuf[i, pl.ds(0, 16)] = jnp.zeros((16,), jnp.float32)
        return 0

    lax.fori_loop(0, ZR, zfill, 0)

    for r in range(RPT // ZR):
        pltpu.sync_copy(zbuf, acc.at[pl.ds(sid * RPT + r * ZR, ZR)])
    drem = RPT - (RPT // ZR) * ZR
    if drem:
        pltpu.sync_copy(zbuf.at[pl.ds(0, drem)],
                        acc.at[pl.ds(sid * RPT + RPT - drem, drem)])
    plsc.subcore_barrier()

    def body(j, _):
        pltpu.sync_copy(ones_v, acc.at[dst_v.at[j]], add=True)
        return 0

    lax.fori_loop(0, KCH, body, 0)
    plsc.subcore_barrier()
    pltpu.sync_copy(acc.at[pl.ds(sid * RPT, RPT)],
                    out_hbm.at[pl.ds(cid * NP + sid * RPT, RPT)])


def _dinv_block(dr):
    deg = dr[0, :, 0:1] + dr[1, :, 0:1] + 1.0
    return lax.rsqrt(deg)


def _g1_body(xr, wr, dr, gr):
    dinv = _dinv_block(dr)
    hw = jnp.dot(xr[...], wr[...], preferred_element_type=jnp.float32,
                 precision=lax.Precision.HIGHEST)
    gr[...] = hw * dinv


def _hidden_body(sr, gr, dr, wr, b1r, scr, bir, outr):
    dinv = _dinv_block(dr)
    agg = (sr[0] + sr[1] + gr[...]) * dinv
    h = (agg + b1r[...]) * scr[...] + bir[...]
    h = jnp.maximum(h, 0.0)
    outr[...] = jnp.dot(h, wr[...], preferred_element_type=jnp.float32,
                        precision=lax.Precision.HIGHEST) * dinv


def _out_body(sr, gr, dr, br, outr):
    dinv = _dinv_block(dr)
    outr[...] = (sr[0] + sr[1] + gr[...]) * dinv + br[...]


def _row_spec(w):
    return pl.BlockSpec((RB, w), lambda i: (i, 0))


def _part_spec(w):
    return pl.BlockSpec((NC, RB, w), lambda i: (0, i, 0))


def _full_spec(shape):
    return pl.BlockSpec(shape, lambda i: tuple(0 for _ in shape))


_GRID = (NP // RB,)

_g1_call = pl.pallas_call(
    _g1_body,
    grid=_GRID,
    in_specs=[_row_spec(D), _full_spec((D, H)), _part_spec(DW)],
    out_specs=_row_spec(H),
    out_shape=jax.ShapeDtypeStruct((NP, H), jnp.float32),
)

_hidden_call = pl.pallas_call(
    _hidden_body,
    grid=_GRID,
    in_specs=[_part_spec(H), _row_spec(H), _part_spec(DW),
              _full_spec((H, CP)), _full_spec((1, H)), _full_spec((1, H)),
              _full_spec((1, H))],
    out_specs=_row_spec(CP),
    out_shape=jax.ShapeDtypeStruct((NP, CP), jnp.float32),
)

_out_call = pl.pallas_call(
    _out_body,
    grid=_GRID,
    in_specs=[_part_spec(CP), _row_spec(CP), _part_spec(DW),
              _full_spec((1, CP))],
    out_specs=_row_spec(CP),
    out_shape=jax.ShapeDtypeStruct((NP, CP), jnp.float32),
)

_seg_h = _seg_sum_kernel(H)
_seg_c = _seg_sum_kernel(CP, tc_tiling=False)


def kernel(x, edge_index, W1, b1, bn1_scale, bn1_bias, W2, b2):
    src = edge_index[0].astype(jnp.int32)
    dst = edge_index[1].astype(jnp.int32)
    fill = jnp.full((EP - E,), N_NODES, jnp.int32)
    src_p = jnp.concatenate([src, fill]).reshape(ECH, CHUNK)
    dst_p = jnp.concatenate([dst, fill]).reshape(ECH, CHUNK)
    x_p = jnp.pad(x, ((0, NP - N_NODES), (0, 0)))
    W2p = jnp.pad(W2, ((0, 0), (0, CP - C)))
    b2p = jnp.pad(b2, ((0, CP - C),)).reshape(1, CP)
    b1r = b1.reshape(1, H)
    scr = bn1_scale.reshape(1, H)
    bir = bn1_bias.reshape(1, H)

    degp = _deg_kernel(dst_p).reshape(NC, NP, DW)
    g1 = _g1_call(x_p, W1, degp)
    s1 = _seg_h(g1, src_p, dst_p).reshape(NC, NP, H)
    g2 = _hidden_call(s1, g1, degp, W2p, b1r, scr, bir)
    s2 = _seg_c(g2, src_p, dst_p).reshape(NC, NP, CP)
    outp = _out_call(s2, g2, degp, b2p)
    return outp[:N_NODES, :C]


# speedup vs baseline: 1.0842x; 1.0611x over previous
import functools

import jax
import jax.numpy as jnp
from jax import lax
from jax.experimental import pallas as pl
from jax.experimental.pallas import tpu as pltpu
from jax.experimental.pallas import tpu_sc as plsc

N_NODES = 10000
NP = 10112
D = 128
H = 128
C = 40
CP = 48
E = 320000
CHUNK = 128
NC, NS = 2, 16
NW = NC * NS
KCH = 80
EP = NW * CHUNK * KCH
ECH = EP // CHUNK
RPT = NP // NS
ZR = 64
DW = 16
RB = 632
K_SLOW = 80
K_FAST = 2 * KCH - K_SLOW
SLOW_CORE = 1
SMAX = K_FAST // 2

_MESH = dict(core_axis_name="c", subcore_axis_name="s")


def _seg_sum_kernel(width, tc_tiling=True):
    @functools.partial(
        pl.kernel,
        out_type=jax.ShapeDtypeStruct((NC * NP, width), jnp.float32),
        mesh=plsc.VectorSubcoreMesh(**_MESH),
        compiler_params=pltpu.CompilerParams(use_tc_tiling_on_sc=tc_tiling),
        scratch_types=[
            pltpu.VMEM((SMAX, CHUNK), jnp.int32),
            pltpu.VMEM((SMAX, CHUNK), jnp.int32),
            pltpu.VMEM((CHUNK, width), jnp.float32),
            pltpu.VMEM((CHUNK, width), jnp.float32),
            pltpu.VMEM_SHARED((NP, width), jnp.float32),
            pltpu.SemaphoreType.DMA,
            pltpu.SemaphoreType.DMA,
        ],
    )
    def k(g_hbm, src_hbm, dst_hbm, out_hbm, src_v, dst_v, buf0, buf1, acc,
          gsem0, gsem1):
        cid = lax.axis_index("c")
        sid = lax.axis_index("s")
        bufs = (buf0, buf1)
        gsems = (gsem0, gsem1)

        slow = cid == SLOW_CORE
        count = jnp.where(slow, K_SLOW, K_FAST)
        scount = count // 2
        tile_base = jnp.where(slow, sid * K_SLOW,
                              NS * K_SLOW + sid * K_FAST)

        def start_gather(b, j):
            pltpu.async_copy(g_hbm.at[src_v.at[j]], bufs[b], gsems[b])

        def wait_gather(b, j):
            pltpu.make_async_copy(g_hbm.at[src_v.at[j]], bufs[b],
                                  gsems[b]).wait()

        def zfill(i, _):
            r = i // (width // 16)
            c = lax.rem(i, width // 16)
            buf0[r, pl.ds(c * 16, 16)] = jnp.zeros((16,), jnp.float32)
            return 0

        lax.fori_loop(0, CHUNK * (width // 16), zfill, 0)

        for r in range(RPT // CHUNK):
            pltpu.sync_copy(buf0, acc.at[pl.ds(sid * RPT + r * CHUNK, CHUNK)])
        rem = RPT - (RPT // CHUNK) * CHUNK
        if rem:
            pltpu.sync_copy(
                buf0.at[pl.ds(0, rem)],
                acc.at[pl.ds(sid * RPT + RPT - rem, rem)])
        plsc.subcore_barrier()

        for s in range(2):
            base = pl.multiple_of(tile_base + s * scount, 8)
            pltpu.sync_copy(src_hbm.at[pl.ds(base, SMAX)], src_v)
            pltpu.sync_copy(dst_hbm.at[pl.ds(base, SMAX)], dst_v)

            start_gather(0, 0)

            def body(i, _):
                for b in range(2):
                    j = 2 * i + b
                    jn = j + 1
                    bn = 1 - b

                    @pl.when(jn < scount)
                    def _():
                        start_gather(bn, jn)

                    wait_gather(b, j)
                    pltpu.sync_copy(bufs[b], acc.at[dst_v.at[j]], add=True)
                return 0

            lax.fori_loop(0, scount // 2, body, 0)

        plsc.subcore_barrier()
        pltpu.sync_copy(acc.at[pl.ds(sid * RPT, RPT)],
                        out_hbm.at[pl.ds(cid * NP + sid * RPT, RPT)])

    return k


@functools.partial(
    pl.kernel,
    out_type=jax.ShapeDtypeStruct((NC * NP, DW), jnp.float32),
    mesh=plsc.VectorSubcoreMesh(**_MESH),
    scratch_types=[
        pltpu.VMEM((KCH, CHUNK), jnp.int32),
        pltpu.VMEM((CHUNK, DW), jnp.float32),
        pltpu.VMEM((ZR, DW), jnp.float32),
        pltpu.VMEM_SHARED((NP, DW), jnp.float32),
        pltpu.SemaphoreType.DMA,
    ],
)
def _deg_kernel(dst_hbm, out_hbm, dst_v, ones_v, zbuf, acc, sem):
    cid = lax.axis_index("c")
    sid = lax.axis_index("s")
    tid = cid * NS + sid
    pltpu.sync_copy(dst_hbm.at[pl.ds(tid * KCH, KCH)], dst_v)

    def ofill(i, _):
        ones_v[i, pl.ds(0, 16)] = jnp.ones((16,), jnp.float32)
        return 0

    lax.fori_loop(0, CHUNK, ofill, 0)

    def zfill(i, _):
        zbuf[i, pl.ds(0, 16)] = jnp.zeros((16,), jnp.float32)
        return 0

    lax.fori_loop(0, ZR, zfill, 0)

    for r in range(RPT // ZR):
        pltpu.sync_copy(zbuf, acc.at[pl.ds(sid * RPT + r * ZR, ZR)])
    drem = RPT - (RPT // ZR) * ZR
    if drem:
        pltpu.sync_copy(zbuf.at[pl.ds(0, drem)],
                        acc.at[pl.ds(sid * RPT + RPT - drem, drem)])
    plsc.subcore_barrier()

    def body(j, _):
        pltpu.sync_copy(ones_v, acc.at[dst_v.at[j]], add=True)
        return 0

    lax.fori_loop(0, KCH, body, 0)
    plsc.subcore_barrier()
    pltpu.sync_copy(acc.at[pl.ds(sid * RPT, RPT)],
                    out_hbm.at[pl.ds(cid * NP + sid * RPT, RPT)])


def _dinv_block(dr):
    deg = dr[0, :, 0:1] + dr[1, :, 0:1] + 1.0
    return lax.rsqrt(deg)


def _g1_body(xr, wr, dr, gr):
    dinv = _dinv_block(dr)
    hw = jnp.dot(xr[...], wr[...], preferred_element_type=jnp.float32,
                 precision=lax.Precision.HIGHEST)
    gr[...] = hw * dinv


def _hidden_body(sr, gr, dr, wr, b1r, scr, bir, outr):
    dinv = _dinv_block(dr)
    agg = (sr[0] + sr[1] + gr[...]) * dinv
    h = (agg + b1r[...]) * scr[...] + bir[...]
    h = jnp.maximum(h, 0.0)
    outr[...] = jnp.dot(h, wr[...], preferred_element_type=jnp.float32,
                        precision=lax.Precision.HIGHEST) * dinv


def _out_body(sr, gr, dr, br, outr):
    dinv = _dinv_block(dr)
    outr[...] = (sr[0] + sr[1] + gr[...]) * dinv + br[...]


def _row_spec(w):
    return pl.BlockSpec((RB, w), lambda i: (i, 0))


def _part_spec(w):
    return pl.BlockSpec((NC, RB, w), lambda i: (0, i, 0))


def _full_spec(shape):
    return pl.BlockSpec(shape, lambda i: tuple(0 for _ in shape))


_GRID = (NP // RB,)

_g1_call = pl.pallas_call(
    _g1_body,
    grid=_GRID,
    in_specs=[_row_spec(D), _full_spec((D, H)), _part_spec(DW)],
    out_specs=_row_spec(H),
    out_shape=jax.ShapeDtypeStruct((NP, H), jnp.float32),
)

_hidden_call = pl.pallas_call(
    _hidden_body,
    grid=_GRID,
    in_specs=[_part_spec(H), _row_spec(H), _part_spec(DW),
              _full_spec((H, CP)), _full_spec((1, H)), _full_spec((1, H)),
              _full_spec((1, H))],
    out_specs=_row_spec(CP),
    out_shape=jax.ShapeDtypeStruct((NP, CP), jnp.float32),
)

_out_call = pl.pallas_call(
    _out_body,
    grid=_GRID,
    in_specs=[_part_spec(CP), _row_spec(CP), _part_spec(DW),
              _full_spec((1, CP))],
    out_specs=_row_spec(CP),
    out_shape=jax.ShapeDtypeStruct((NP, CP), jnp.float32),
)

_seg_h = _seg_sum_kernel(H)
_seg_c = _seg_sum_kernel(CP, tc_tiling=False)


def kernel(x, edge_index, W1, b1, bn1_scale, bn1_bias, W2, b2):
    src = edge_index[0].astype(jnp.int32)
    dst = edge_index[1].astype(jnp.int32)
    fill = jnp.full((EP - E,), N_NODES, jnp.int32)
    src_p = jnp.concatenate([src, fill]).reshape(ECH, CHUNK)
    dst_p = jnp.concatenate([dst, fill]).reshape(ECH, CHUNK)
    x_p = jnp.pad(x, ((0, NP - N_NODES), (0, 0)))
    W2p = jnp.pad(W2, ((0, 0), (0, CP - C)))
    b2p = jnp.pad(b2, ((0, CP - C),)).reshape(1, CP)
    b1r = b1.reshape(1, H)
    scr = bn1_scale.reshape(1, H)
    bir = bn1_bias.reshape(1, H)

    degp = _deg_kernel(dst_p).reshape(NC, NP, DW)
    g1 = _g1_call(x_p, W1, degp)
    s1 = _seg_h(g1, src_p, dst_p).reshape(NC, NP, H)
    g2 = _hidden_call(s1, g1, degp, W2p, b1r, scr, bir)
    s2 = _seg_c(g2, src_p, dst_p).reshape(NC, NP, CP)
    outp = _out_call(s2, g2, degp, b2p)
    return outp[:N_NODES, :C]


# speedup vs baseline: 1.2844x; 1.1846x over previous
import functools

import jax
import jax.numpy as jnp
from jax import lax
from jax.experimental import pallas as pl
from jax.experimental.pallas import tpu as pltpu
from jax.experimental.pallas import tpu_sc as plsc

N_NODES = 10000
NP = 10240
D = 128
H = 128
C = 40
CP = 48
E = 320000
CHUNK = 128
NC, NS = 2, 16
NW = NC * NS
KCH = 80
EP = NW * CHUNK * KCH
ECH = EP // CHUNK
RPT = NP // NS
ZR = 64
DW = 16
RB = 1024
SSTAGE = 40

_MESH = dict(core_axis_name="c", subcore_axis_name="s")


def _seg_sum_kernel(width, tc_tiling=True):
    @functools.partial(
        pl.kernel,
        out_type=jax.ShapeDtypeStruct((NC * NP, width), jnp.float32),
        mesh=plsc.VectorSubcoreMesh(**_MESH),
        compiler_params=pltpu.CompilerParams(use_tc_tiling_on_sc=tc_tiling),
        scratch_types=[
            pltpu.VMEM((SSTAGE, CHUNK), jnp.int32),
            pltpu.VMEM((SSTAGE, CHUNK), jnp.int32),
            pltpu.VMEM((CHUNK, width), jnp.float32),
            pltpu.VMEM((CHUNK, width), jnp.float32),
            pltpu.VMEM_SHARED((NP, width), jnp.float32),
            pltpu.SemaphoreType.DMA,
            pltpu.SemaphoreType.DMA,
            pltpu.SemaphoreType.DMA,
            pltpu.SemaphoreType.DMA,
        ],
    )
    def k(g_hbm, src_hbm, dst_hbm, out_hbm, src_v, dst_v, buf0, buf1, acc,
          gsem0, gsem1, ssem0, ssem1):
        cid = lax.axis_index("c")
        sid = lax.axis_index("s")
        tid = cid * NS + sid
        bufs = (buf0, buf1)
        gsems = (gsem0, gsem1)

        def wait_gather(b, j):
            pltpu.make_async_copy(g_hbm.at[src_v.at[j]], bufs[b],
                                  gsems[b]).wait()

        def zfill(i, _):
            r = i // (width // 16)
            c = lax.rem(i, width // 16)
            buf0[r, pl.ds(c * 16, 16)] = jnp.zeros((16,), jnp.float32)
            return 0

        lax.fori_loop(0, CHUNK * (width // 16), zfill, 0)

        def zinit(r, _):
            pltpu.sync_copy(buf0, acc.at[pl.ds(sid * RPT + r * CHUNK, CHUNK)])
            return 0

        lax.fori_loop(0, RPT // CHUNK, zinit, 0)
        plsc.subcore_barrier()

        for s in range(KCH // SSTAGE):
            pltpu.sync_copy(src_hbm.at[pl.ds(tid * KCH + s * SSTAGE, SSTAGE)],
                            src_v)
            pltpu.sync_copy(dst_hbm.at[pl.ds(tid * KCH + s * SSTAGE, SSTAGE)],
                            dst_v)
            pltpu.async_copy(g_hbm.at[src_v.at[0]], buf0, gsem0)

            def body(i, _):
                for b in range(2):
                    j = 2 * i + b
                    jn = j + 1
                    bn = 1 - b

                    @pl.when(jn < SSTAGE)
                    def _():
                        pltpu.async_copy(g_hbm.at[src_v.at[jn]], bufs[bn],
                                         gsems[bn])

                    wait_gather(b, j)
                    pltpu.sync_copy(bufs[b], acc.at[dst_v.at[j]], add=True)
                return 0

            lax.fori_loop(0, SSTAGE // 2, body, 0)

        plsc.subcore_barrier()
        pltpu.sync_copy(acc.at[pl.ds(sid * RPT, RPT)],
                        out_hbm.at[pl.ds(cid * NP + sid * RPT, RPT)])

    return k


@functools.partial(
    pl.kernel,
    out_type=jax.ShapeDtypeStruct((NC * NP, DW), jnp.float32),
    mesh=plsc.VectorSubcoreMesh(**_MESH),
    scratch_types=[
        pltpu.VMEM((KCH, CHUNK), jnp.int32),
        pltpu.VMEM((CHUNK, DW), jnp.float32),
        pltpu.VMEM((ZR, DW), jnp.float32),
        pltpu.VMEM_SHARED((NP, DW), jnp.float32),
        pltpu.SemaphoreType.DMA,
    ],
)
def _deg_kernel(dst_hbm, out_hbm, dst_v, ones_v, zbuf, acc, sem):
    cid = lax.axis_index("c")
    sid = lax.axis_index("s")
    tid = cid * NS + sid
    pltpu.sync_copy(dst_hbm.at[pl.ds(tid * KCH, KCH)], dst_v)

    def ofill(i, _):
        ones_v[i, pl.ds(0, 16)] = jnp.ones((16,), jnp.float32)
        return 0

    lax.fori_loop(0, CHUNK, ofill, 0)

    def zfill(i, _):
        zbuf[i, pl.ds(0, 16)] = jnp.zeros((16,), jnp.float32)
        return 0

    lax.fori_loop(0, ZR, zfill, 0)

    for r in range(RPT // ZR):
        pltpu.sync_copy(zbuf, acc.at[pl.ds(sid * RPT + r * ZR, ZR)])
    drem = RPT - (RPT // ZR) * ZR
    if drem:
        pltpu.sync_copy(zbuf.at[pl.ds(0, drem)],
                        acc.at[pl.ds(sid * RPT + RPT - drem, drem)])
    plsc.subcore_barrier()

    def body(j, _):
        pltpu.sync_copy(ones_v, acc.at[dst_v.at[j]], add=True)
        return 0

    lax.fori_loop(0, KCH, body, 0)
    plsc.subcore_barrier()
    pltpu.sync_copy(acc.at[pl.ds(sid * RPT, RPT)],
                    out_hbm.at[pl.ds(cid * NP + sid * RPT, RPT)])


def _dinv_block(dr):
    deg = dr[0, :, 0:1] + dr[1, :, 0:1] + 1.0
    return lax.rsqrt(deg)


def _g1_body(xr, wr, dr, gr):
    dinv = _dinv_block(dr)
    hw = jnp.dot(xr[...], wr[...], preferred_element_type=jnp.float32,
                 precision=lax.Precision.HIGHEST)
    gr[...] = hw * dinv


def _hidden_body(sr, gr, dr, wr, b1r, scr, bir, outr):
    dinv = _dinv_block(dr)
    agg = (sr[0] + sr[1] + gr[...]) * dinv
    h = (agg + b1r[...]) * scr[...] + bir[...]
    h = jnp.maximum(h, 0.0)
    outr[...] = jnp.dot(h, wr[...], preferred_element_type=jnp.float32,
                        precision=lax.Precision.HIGHEST) * dinv


def _out_body(sr, gr, dr, br, outr):
    dinv = _dinv_block(dr)
    outr[...] = (sr[0] + sr[1] + gr[...]) * dinv + br[...]


def _row_spec(w):
    return pl.BlockSpec((RB, w), lambda i: (i, 0))


def _part_spec(w):
    return pl.BlockSpec((NC, RB, w), lambda i: (0, i, 0))


def _full_spec(shape):
    return pl.BlockSpec(shape, lambda i: tuple(0 for _ in shape))


_GRID = (NP // RB,)

_g1_call = pl.pallas_call(
    _g1_body,
    grid=_GRID,
    in_specs=[_row_spec(D), _full_spec((D, H)), _part_spec(DW)],
    out_specs=_row_spec(H),
    out_shape=jax.ShapeDtypeStruct((NP, H), jnp.float32),
)

_hidden_call = pl.pallas_call(
    _hidden_body,
    grid=_GRID,
    in_specs=[_part_spec(H), _row_spec(H), _part_spec(DW),
              _full_spec((H, CP)), _full_spec((1, H)), _full_spec((1, H)),
              _full_spec((1, H))],
    out_specs=_row_spec(CP),
    out_shape=jax.ShapeDtypeStruct((NP, CP), jnp.float32),
)

_out_call = pl.pallas_call(
    _out_body,
    grid=_GRID,
    in_specs=[_part_spec(CP), _row_spec(CP), _part_spec(DW),
              _full_spec((1, CP))],
    out_specs=_row_spec(CP),
    out_shape=jax.ShapeDtypeStruct((NP, CP), jnp.float32),
)

_seg_h = _seg_sum_kernel(H)
_seg_c = _seg_sum_kernel(CP, tc_tiling=False)


def kernel(x, edge_index, W1, b1, bn1_scale, bn1_bias, W2, b2):
    src = edge_index[0].astype(jnp.int32)
    dst = edge_index[1].astype(jnp.int32)
    fill = jnp.full((EP - E,), N_NODES, jnp.int32)
    src_p = jnp.concatenate([src, fill]).reshape(ECH, CHUNK)
    dst_p = jnp.concatenate([dst, fill]).reshape(ECH, CHUNK)
    x_p = jnp.pad(x, ((0, NP - N_NODES), (0, 0)))
    W2p = jnp.pad(W2, ((0, 0), (0, CP - C)))
    b2p = jnp.pad(b2, ((0, CP - C),)).reshape(1, CP)
    b1r = b1.reshape(1, H)
    scr = bn1_scale.reshape(1, H)
    bir = bn1_bias.reshape(1, H)

    degp = _deg_kernel(dst_p).reshape(NC, NP, DW)
    g1 = _g1_call(x_p, W1, degp)
    s1 = _seg_h(g1, src_p, dst_p).reshape(NC, NP, H)
    g2 = _hidden_call(s1, g1, degp, W2p, b1r, scr, bir)
    s2 = _seg_c(g2, src_p, dst_p).reshape(NC, NP, CP)
    outp = _out_call(s2, g2, degp, b2p)
    return outp[:N_NODES, :C]
